# baseline (speedup 1.0000x reference)
import dataclasses
import functools

import jax
import jax.numpy as jnp
from jax import lax
from jax.experimental import pallas as pl
from jax.experimental.pallas import tpu as pltpu
from jax.experimental.pallas import tpu_sc as plsc

_LANES = 16


def _sc_compiler_params():
    cp = pltpu.CompilerParams()
    if "needs_layout_passes" in pltpu.CompilerParams.__dataclass_fields__:
        cp = dataclasses.replace(cp, needs_layout_passes=False)
    return cp


def _gather_coeffs_sc(t, tab_a, tab_s):
    B = t.shape[0]
    T = tab_a.shape[0]
    mesh = plsc.ScalarSubcoreMesh(axis_name="c", num_cores=2)

    @functools.partial(
        pl.kernel,
        out_type=(
            jax.ShapeDtypeStruct((B,), jnp.float32),
            jax.ShapeDtypeStruct((B,), jnp.float32),
        ),
        mesh=mesh,
        scratch_types=[
            pltpu.SMEM((B,), jnp.int32),
            pltpu.SMEM((T,), jnp.float32),
            pltpu.SMEM((B,), jnp.float32),
            pltpu.SemaphoreType.DMA,
            pltpu.SemaphoreType.DMA,
        ],
        compiler_params=_sc_compiler_params(),
    )
    def gather_kernel(t_hbm, ta_hbm, ts_hbm, oa_hbm, os_hbm,
                      idx_s, tab_s_ref, out_s, sem0, sem1):
        cid = lax.axis_index("c")
        c0 = pltpu.async_copy(t_hbm, idx_s, sem0)

        @pl.when(cid == 0)
        def _():
            pltpu.async_copy(ta_hbm, tab_s_ref, sem1).wait()

        @pl.when(cid == 1)
        def _():
            pltpu.async_copy(ts_hbm, tab_s_ref, sem1).wait()

        c0.wait()

        @pl.loop(0, B)
        def _(i):
            out_s[i] = tab_s_ref[idx_s[i]]

        @pl.when(cid == 0)
        def _():
            pltpu.async_copy(out_s, oa_hbm, sem1).wait()

        @pl.when(cid == 1)
        def _():
            pltpu.async_copy(out_s, os_hbm, sem1).wait()

    return gather_kernel(t, tab_a, tab_s)


def _combine_body(a_ref, s_ref, x_ref, n_ref, o_ref):
    o_ref[...] = a_ref[...] * x_ref[...] + s_ref[...] * n_ref[...]


def _combine_tc(xt, nt, a, s, h_block):
    C, H, W, B = xt.shape
    HB = h_block
    blk = (C, HB, W, B)
    idx = lambda j: (0, j, 0, 0)
    cidx = lambda j: (0,)
    return pl.pallas_call(
        _combine_body,
        grid=(H // HB,),
        in_specs=[
            pl.BlockSpec((B,), cidx),
            pl.BlockSpec((B,), cidx),
            pl.BlockSpec(blk, idx),
            pl.BlockSpec(blk, idx),
        ],
        out_specs=pl.BlockSpec(blk, idx),
        out_shape=jax.ShapeDtypeStruct((C, H, W, B), jnp.float32),
        compiler_params=pltpu.CompilerParams(
            dimension_semantics=("arbitrary",),
        ),
    )(a, s, xt, nt)


def kernel(x0, t, noise, sqrt_alphas_bar, sqrt_one_minus_alphas_bar):
    a, s = _gather_coeffs_sc(t, sqrt_alphas_bar, sqrt_one_minus_alphas_bar)
    xt = jnp.transpose(x0, (1, 2, 3, 0))
    nt = jnp.transpose(noise, (1, 2, 3, 0))
    out_t = _combine_tc(xt, nt, a, s, h_block=16)
    return jnp.transpose(out_t, (3, 0, 1, 2))
